# baseline (speedup 1.0000x reference)
import math

import jax
import jax.numpy as jnp
from jax import lax
from jax.experimental import pallas as pl
from jax.experimental.pallas import tpu as pltpu
from jax.experimental.pallas import tpu_sc as plsc

_DIM = 64
_SCALE = math.sqrt(_DIM)
_NC = 2
_NS = 16
_NW = _NC * _NS
_LANES = 16
_B, _S = 4096, 200
_ST, _BT = _S // 8, _B // 128
_RH = 512
_NU = _ST * _BT * 2
_UPW = _NU // _NW
_TILE = 1024


def _make_kernel():
    mesh = plsc.VectorSubcoreMesh(core_axis_name="c", subcore_axis_name="s")

    def body(xf, table, outf,
             idx0, idx1, rows0, rows1, stg0, stg1, stg2, stg3,
             isem0, isem1, gsem0, gsem1, ssem0, ssem1, ssem2, ssem3):
        wid = lax.axis_index("s") * _NC + lax.axis_index("c")
        idxb = (idx0, idx1)
        rowsb = (rows0, rows1)
        isem = (isem0, isem1)
        gsem = (gsem0, gsem1)
        stg = (stg0, stg1, stg2, stg3)
        ssem = (ssem0, ssem1, ssem2, ssem3)
        iota = lax.iota(jnp.int32, _LANES)
        c0 = lax.shift_left(lax.shift_right_logical(iota, 3), 10) + lax.shift_left(
            lax.bitwise_and(iota, 7), 7
        )

        def idx_copy(u, p):
            h = wid * _UPW + u
            return pltpu.make_async_copy(
                xf.at[pl.ds(h * _RH, _RH)], idxb[p], isem[p]
            )

        def gather(p):
            return pltpu.make_async_copy(table.at[idxb[p]], rowsb[p], gsem[p])

        def store_dmas(u, sl):
            h = wid * _UPW + u
            st = h // (_BT * 2)
            rem = h % (_BT * 2)
            bt = rem // 2
            half = rem % 2
            s_glob = st * 8 + half * 4 + sl
            woff = s_glob * (8 * _BT * _TILE) + bt * _TILE
            return [
                pltpu.make_async_copy(
                    stg[sl].at[pl.ds(dt * _TILE, _TILE)],
                    outf.at[pl.ds(woff + dt * (_BT * _TILE), _TILE)],
                    ssem[sl],
                )
                for dt in range(8)
            ]

        idx_copy(0, 0).start()
        idx_copy(1, 1).start()
        idx_copy(0, 0).wait()
        gather(0).start()

        @pl.loop(0, _UPW, step=2)
        def _ring(g):
            for p in range(2):
                u = g + p
                gather(p).wait()

                @pl.when(u + 2 < _UPW)
                def _():
                    idx_copy(u + 2, p).start()

                @pl.when(u + 1 < _UPW)
                def _():
                    idx_copy(u + 1, 1 - p).wait()
                    gather(1 - p).start()

                rows = rowsb[p]
                for sl in range(4):
                    @pl.when(u >= 1)
                    def _():
                        for d in store_dmas(u - 1, sl):
                            d.wait()

                    @pl.loop(0, 128, step=4)
                    def _b(b0):
                        for bb in range(4):
                            b = b0 + bb
                            r = sl * 128 + b
                            for k in range(_DIM // _LANES):
                                v = rows[r, pl.ds(k * _LANES, _LANES)]
                                plsc.store_scatter(
                                    stg[sl],
                                    [c0 + (2 * k * _TILE + b)],
                                    v * _SCALE,
                                )

                    for d in store_dmas(u, sl):
                        d.start()

        for sl in range(4):
            for d in store_dmas(_UPW - 1, sl):
                d.wait()

    return pl.kernel(
        body,
        out_type=jax.ShapeDtypeStruct((_S * 8 * _BT * _TILE,), jnp.float32),
        mesh=mesh,
        scratch_types=[
            pltpu.VMEM((_RH,), jnp.int32),
            pltpu.VMEM((_RH,), jnp.int32),
            pltpu.VMEM((_RH, _DIM), jnp.float32),
            pltpu.VMEM((_RH, _DIM), jnp.float32),
            pltpu.VMEM((8 * _TILE,), jnp.float32),
            pltpu.VMEM((8 * _TILE,), jnp.float32),
            pltpu.VMEM((8 * _TILE,), jnp.float32),
            pltpu.VMEM((8 * _TILE,), jnp.float32),
            pltpu.SemaphoreType.DMA,
            pltpu.SemaphoreType.DMA,
            pltpu.SemaphoreType.DMA,
            pltpu.SemaphoreType.DMA,
            pltpu.SemaphoreType.DMA,
            pltpu.SemaphoreType.DMA,
            pltpu.SemaphoreType.DMA,
            pltpu.SemaphoreType.DMA,
        ],
        compiler_params=pltpu.CompilerParams(
            use_tc_tiling_on_sc=False, needs_layout_passes=False
        ),
    )


def kernel(x, table):
    xf = (x.astype(jnp.int32).T
          .reshape(_ST, 8, _BT, 128).transpose(0, 2, 1, 3).reshape(-1))
    outf = _make_kernel()(xf, table)
    return (outf.reshape(_S, 8, _BT, 8, 128)
            .transpose(2, 4, 0, 1, 3).reshape(_B, _S, _DIM))


# speedup vs baseline: 1.3175x; 1.3175x over previous
import math

import jax
import jax.numpy as jnp
from jax import lax
from jax.experimental import pallas as pl
from jax.experimental.pallas import tpu as pltpu
from jax.experimental.pallas import tpu_sc as plsc

_DIM = 64
_SCALE = math.sqrt(_DIM)
_NC = 2
_NS = 16
_NW = _NC * _NS
_LANES = 16
_B, _S = 4096, 200
_ST, _BT = _S // 8, _B // 128
_RH = 512
_NU = _ST * _BT * 2
_UPW = _NU // _NW
_TILE = 1024


def _make_kernel():
    mesh = plsc.VectorSubcoreMesh(core_axis_name="c", subcore_axis_name="s")

    def body(xf, table, outf,
             idx0, idx1, rows0, rows1, stg0, stg1, stg2, stg3,
             isem0, isem1, gsem0, gsem1, ssem0, ssem1, ssem2, ssem3):
        wid = lax.axis_index("s") * _NC + lax.axis_index("c")
        idxb = (idx0, idx1)
        rowsb = (rows0, rows1)
        isem = (isem0, isem1)
        gsem = (gsem0, gsem1)
        stg = (stg0, stg1, stg2, stg3)
        ssem = (ssem0, ssem1, ssem2, ssem3)
        iota = lax.iota(jnp.int32, _LANES)
        c0 = lax.shift_left(lax.shift_right_logical(iota, 3), 10) + lax.shift_left(
            lax.bitwise_and(iota, 7), 7
        )

        def idx_copy(u, p):
            h = wid * _UPW + u
            return pltpu.make_async_copy(
                xf.at[pl.ds(h * _RH, _RH)], idxb[p], isem[p]
            )

        def gather(p):
            return pltpu.make_async_copy(table.at[idxb[p]], rowsb[p], gsem[p])

        def store_dmas(u, sl):
            h = wid * _UPW + u
            st = h // (_BT * 2)
            rem = h % (_BT * 2)
            bt = rem // 2
            half = rem % 2
            s_glob = st * 8 + half * 4 + sl
            woff = s_glob * (8 * _BT * _TILE) + bt * _TILE
            return [
                pltpu.make_async_copy(
                    stg[sl].at[pl.ds(dt * _TILE, _TILE)],
                    outf.at[pl.ds(woff + dt * (_BT * _TILE), _TILE)],
                    ssem[sl],
                )
                for dt in range(8)
            ]

        idx_copy(0, 0).start()
        idx_copy(1, 1).start()
        idx_copy(0, 0).wait()
        gather(0).start()

        @pl.loop(0, _UPW, step=2)
        def _ring(g):
            for p in range(2):
                u = g + p
                gather(p).wait()

                @pl.when(u + 2 < _UPW)
                def _():
                    idx_copy(u + 2, p).start()

                @pl.when(u + 1 < _UPW)
                def _():
                    idx_copy(u + 1, 1 - p).wait()
                    gather(1 - p).start()

                rows = rowsb[p]
                for sl in range(4):
                    @pl.when(u >= 1)
                    def _():
                        for d in store_dmas(u - 1, sl):
                            d.wait()

                    @plsc.parallel_loop(0, 128, unroll=8)
                    def _b(b):
                        r = sl * 128 + b
                        for k in range(_DIM // _LANES):
                            v = rows[r, pl.ds(k * _LANES, _LANES)]
                            plsc.store_scatter(
                                stg[sl],
                                [c0 + (2 * k * _TILE + b)],
                                v * _SCALE,
                            )

                    for d in store_dmas(u, sl):
                        d.start()

        for sl in range(4):
            for d in store_dmas(_UPW - 1, sl):
                d.wait()

    return pl.kernel(
        body,
        out_type=jax.ShapeDtypeStruct((_S * 8 * _BT * _TILE,), jnp.float32),
        mesh=mesh,
        scratch_types=[
            pltpu.VMEM((_RH,), jnp.int32),
            pltpu.VMEM((_RH,), jnp.int32),
            pltpu.VMEM((_RH, _DIM), jnp.float32),
            pltpu.VMEM((_RH, _DIM), jnp.float32),
            pltpu.VMEM((8 * _TILE,), jnp.float32),
            pltpu.VMEM((8 * _TILE,), jnp.float32),
            pltpu.VMEM((8 * _TILE,), jnp.float32),
            pltpu.VMEM((8 * _TILE,), jnp.float32),
            pltpu.SemaphoreType.DMA,
            pltpu.SemaphoreType.DMA,
            pltpu.SemaphoreType.DMA,
            pltpu.SemaphoreType.DMA,
            pltpu.SemaphoreType.DMA,
            pltpu.SemaphoreType.DMA,
            pltpu.SemaphoreType.DMA,
            pltpu.SemaphoreType.DMA,
        ],
        compiler_params=pltpu.CompilerParams(
            use_tc_tiling_on_sc=False, needs_layout_passes=False
        ),
    )


def kernel(x, table):
    xf = (x.astype(jnp.int32).T
          .reshape(_ST, 8, _BT, 128).transpose(0, 2, 1, 3).reshape(-1))
    outf = _make_kernel()(xf, table)
    return (outf.reshape(_S, 8, _BT, 8, 128)
            .transpose(2, 4, 0, 1, 3).reshape(_B, _S, _DIM))
